# baseline (speedup 1.0000x reference)
import functools

import jax
import jax.numpy as jnp
from jax import lax
from jax.experimental import pallas as pl
from jax.experimental.pallas import tpu as pltpu
from jax.experimental.pallas import tpu_sc as plsc

N_NODES = 10000
N_PAD = 10240
D = 128
N_EDGES = 320000
NC = 2
NS = 16
NW = NC * NS
CHUNK = 96
NCHUNKS = 108
E_PAD = NW * NCHUNKS * CHUNK
ROWS_PER_TILE = N_PAD // NS
NROWBUF = 3
NIDXBUF = 4
UNROLL = 12


def _sc_agg_body(x_hbm, ei_hbm, zero_hbm, out_hbm,
                 acc_sh, rows0, rows1, rows2, ib0, ib1, ib2, ib3,
                 gsem0, gsem1, gsem2, isem0, isem1, isem2, isem3):
    c = lax.axis_index("c")
    s = lax.axis_index("s")
    r0 = s * ROWS_PER_TILE
    pltpu.sync_copy(zero_hbm.at[pl.ds(r0, ROWS_PER_TILE)],
                    acc_sh.at[pl.ds(r0, ROWS_PER_TILE)])
    wid = c * NS + s

    rows = (rows0, rows1, rows2)
    gsem = (gsem0, gsem1, gsem2)
    ib = (ib0, ib1, ib2, ib3)
    isem = (isem0, isem1, isem2, isem3)

    def wait_gather(k):
        pltpu.make_async_copy(x_hbm.at[ib0.at[0]], rows[k], gsem[k]).wait()

    def wait_idx(k):
        pltpu.make_async_copy(ei_hbm.at[wid, 0], ib[k], isem[k]).wait()

    pltpu.sync_copy(ei_hbm.at[wid, 0], ib0)
    pltpu.async_copy(ei_hbm.at[wid, 1], ib1, isem1)
    pltpu.async_copy(ei_hbm.at[wid, 2], ib2, isem2)
    pltpu.async_copy(ei_hbm.at[wid, 3], ib3, isem3)
    plsc.subcore_barrier()
    pltpu.async_copy(x_hbm.at[ib0.at[0]], rows0, gsem0)
    wait_idx(1)
    pltpu.async_copy(x_hbm.at[ib1.at[0]], rows1, gsem1)

    def body(t, carry):
        for b in range(UNROLL):
            i = t * UNROLL + b
            rb = b % NROWBUF
            rb2 = (b + 2) % NROWBUF
            ic = b % NIDXBUF
            ic2 = (b + 2) % NIDXBUF
            wait_gather(rb)
            wait_idx(ic2)
            pltpu.async_copy(x_hbm.at[ib[ic2].at[0]], rows[rb2], gsem[rb2])
            pltpu.sync_copy(rows[rb], acc_sh.at[ib[ic].at[1]], add=True)
            nx4 = jnp.minimum(i + 4, NCHUNKS - 1)
            pltpu.async_copy(ei_hbm.at[wid, nx4], ib[ic], isem[ic])
        return carry

    lax.fori_loop(0, NCHUNKS // UNROLL, body, 0)
    wait_gather(NCHUNKS % NROWBUF)
    wait_gather((NCHUNKS + 1) % NROWBUF)
    wait_idx((NCHUNKS - 2) % NIDXBUF)
    wait_idx((NCHUNKS - 1) % NIDXBUF)
    plsc.subcore_barrier()
    pltpu.sync_copy(acc_sh.at[pl.ds(r0, ROWS_PER_TILE)],
                    out_hbm.at[c, pl.ds(r0, ROWS_PER_TILE)])


_sc_agg = functools.partial(
    pl.kernel,
    mesh=plsc.VectorSubcoreMesh(core_axis_name="c", subcore_axis_name="s"),
    out_type=jax.ShapeDtypeStruct((NC, N_PAD, D), jnp.float32),
    scratch_types=[
        pltpu.VMEM_SHARED((N_PAD, D), jnp.float32),
        pltpu.VMEM((CHUNK, D), jnp.float32),
        pltpu.VMEM((CHUNK, D), jnp.float32),
        pltpu.VMEM((CHUNK, D), jnp.float32),
        pltpu.VMEM((2, CHUNK), jnp.int32),
        pltpu.VMEM((2, CHUNK), jnp.int32),
        pltpu.VMEM((2, CHUNK), jnp.int32),
        pltpu.VMEM((2, CHUNK), jnp.int32),
        pltpu.SemaphoreType.DMA,
        pltpu.SemaphoreType.DMA,
        pltpu.SemaphoreType.DMA,
        pltpu.SemaphoreType.DMA,
        pltpu.SemaphoreType.DMA,
        pltpu.SemaphoreType.DMA,
        pltpu.SemaphoreType.DMA,
    ],
)(_sc_agg_body)


BLK = 1024


def _tc_linear_body(p_ref, w_ref, b_ref, o_ref):
    agg = p_ref[0] + p_ref[1]
    o_ref[...] = lax.dot_general(
        agg, w_ref[...], (((1,), (1,)), ((), ())),
        preferred_element_type=jnp.float32) + b_ref[...]


def _tc_linear(partials, W, b):
    return pl.pallas_call(
        _tc_linear_body,
        grid=(N_PAD // BLK,),
        in_specs=[
            pl.BlockSpec((NC, BLK, D), lambda i: (0, i, 0)),
            pl.BlockSpec((D, D), lambda i: (0, 0)),
            pl.BlockSpec((1, D), lambda i: (0, 0)),
        ],
        out_specs=pl.BlockSpec((BLK, D), lambda i: (i, 0)),
        out_shape=jax.ShapeDtypeStruct((N_PAD, D), jnp.float32),
    )(partials, W, b.reshape(1, D))


def kernel(x, edge_index, W, b):
    src = edge_index[0].astype(jnp.int32)
    dst = edge_index[1].astype(jnp.int32)
    npad = E_PAD - N_EDGES
    src = jnp.concatenate([src, jnp.zeros((npad,), jnp.int32)])
    dst = jnp.concatenate(
        [dst, N_NODES + (jnp.arange(npad, dtype=jnp.int32) % (N_PAD - N_NODES))])
    src3 = src.reshape(NW, NCHUNKS, 1, CHUNK)
    dst3 = dst.reshape(NW, NCHUNKS, 1, CHUNK)
    ei4 = jnp.concatenate([src3, dst3], axis=2)
    zero = jnp.zeros((N_PAD, D), jnp.float32)
    partials = _sc_agg(x, ei4, zero)
    return _tc_linear(partials, W, b)[:N_NODES]


# speedup vs baseline: 1.2561x; 1.2561x over previous
import functools

import jax
import jax.numpy as jnp
from jax import lax
from jax.experimental import pallas as pl
from jax.experimental.pallas import tpu as pltpu
from jax.experimental.pallas import tpu_sc as plsc

N_NODES = 10000
N_PAD = 10240
D = 128
N_EDGES = 320000
NC = 2
NS = 16
NW = NC * NS
CHUNK = 128
NCHUNKS = 80
E_PAD = NW * NCHUNKS * CHUNK
ROWS_PER_TILE = N_PAD // NS


def _sc_agg_body(x_hbm, ei_hbm, zero_hbm, out_hbm,
                 acc_sh, rows, ib, gsem):
    c = lax.axis_index("c")
    s = lax.axis_index("s")
    r0 = s * ROWS_PER_TILE
    pltpu.sync_copy(zero_hbm.at[pl.ds(r0, ROWS_PER_TILE)],
                    acc_sh.at[pl.ds(r0, ROWS_PER_TILE)])
    wid = c * NS + s
    plsc.subcore_barrier()

    def body(i, carry):
        pltpu.sync_copy(ei_hbm.at[wid, i], ib)
        pltpu.async_copy(x_hbm.at[ib.at[0]], rows, gsem).wait()
        pltpu.sync_copy(rows, acc_sh.at[ib.at[1]], add=True)
        return carry

    lax.fori_loop(0, NCHUNKS, body, 0)
    plsc.subcore_barrier()
    pltpu.sync_copy(acc_sh.at[pl.ds(r0, ROWS_PER_TILE)],
                    out_hbm.at[c, pl.ds(r0, ROWS_PER_TILE)])


_sc_agg = functools.partial(
    pl.kernel,
    mesh=plsc.VectorSubcoreMesh(core_axis_name="c", subcore_axis_name="s"),
    out_type=jax.ShapeDtypeStruct((NC, N_PAD, D), jnp.float32),
    scratch_types=[
        pltpu.VMEM_SHARED((N_PAD, D), jnp.float32),
        pltpu.VMEM((CHUNK, D), jnp.float32),
        pltpu.VMEM((2, CHUNK), jnp.int32),
        pltpu.SemaphoreType.DMA,
    ],
)(_sc_agg_body)


BLK = 1024


def _tc_linear_body(p_ref, w_ref, b_ref, o_ref):
    agg = p_ref[0] + p_ref[1]
    o_ref[...] = lax.dot_general(
        agg, w_ref[...], (((1,), (1,)), ((), ())),
        preferred_element_type=jnp.float32) + b_ref[...]


def _tc_linear(partials, W, b):
    return pl.pallas_call(
        _tc_linear_body,
        grid=(N_PAD // BLK,),
        in_specs=[
            pl.BlockSpec((NC, BLK, D), lambda i: (0, i, 0)),
            pl.BlockSpec((D, D), lambda i: (0, 0)),
            pl.BlockSpec((1, D), lambda i: (0, 0)),
        ],
        out_specs=pl.BlockSpec((BLK, D), lambda i: (i, 0)),
        out_shape=jax.ShapeDtypeStruct((N_PAD, D), jnp.float32),
    )(partials, W, b.reshape(1, D))


def kernel(x, edge_index, W, b):
    src = edge_index[0].astype(jnp.int32)
    dst = edge_index[1].astype(jnp.int32)
    npad = E_PAD - N_EDGES
    src = jnp.concatenate([src, jnp.zeros((npad,), jnp.int32)])
    dst = jnp.concatenate(
        [dst, N_NODES + (jnp.arange(npad, dtype=jnp.int32) % (N_PAD - N_NODES))])
    src3 = src.reshape(NW, NCHUNKS, 1, CHUNK)
    dst3 = dst.reshape(NW, NCHUNKS, 1, CHUNK)
    ei4 = jnp.concatenate([src3, dst3], axis=2)
    zero = jnp.zeros((N_PAD, D), jnp.float32)
    partials = _sc_agg(x, ei4, zero)
    return _tc_linear(partials, W, b)[:N_NODES]


# speedup vs baseline: 1.3812x; 1.0996x over previous
import functools

import jax
import jax.numpy as jnp
from jax import lax
from jax.experimental import pallas as pl
from jax.experimental.pallas import tpu as pltpu
from jax.experimental.pallas import tpu_sc as plsc

N_NODES = 10000
N_PAD = 10240
D = 128
N_EDGES = 320000
NC = 2
NS = 16
NW = NC * NS
CHUNK = 128
NCHUNKS = 80
E_TILE = N_EDGES // NW
PAD_TILE = NCHUNKS * CHUNK - E_TILE
ROWS_PER_TILE = N_PAD // NS


def _sc_agg_body(x_hbm, src_hbm, dst_hbm, zero_hbm, out_hbm,
                 acc_sh, rows0, rows1,
                 is0, is1, is2, is3, id0, id1, id2, id3,
                 gsem0, gsem1, isem0, isem1, isem2, isem3):
    c = lax.axis_index("c")
    s = lax.axis_index("s")
    r0 = s * ROWS_PER_TILE
    pltpu.sync_copy(zero_hbm.at[pl.ds(r0, ROWS_PER_TILE)],
                    acc_sh.at[pl.ds(r0, ROWS_PER_TILE)])
    wid = c * NS + s

    rows = (rows0, rows1)
    gsem = (gsem0, gsem1)
    isb = (is0, is1, is2, is3)
    idb = (id0, id1, id2, id3)
    isem = (isem0, isem1, isem2, isem3)

    def wait_gather(k):
        pltpu.make_async_copy(x_hbm.at[isb[k]], rows[k % 2],
                              gsem[k % 2]).wait()

    def issue_idx(n, k):
        pltpu.async_copy(src_hbm.at[wid, n], isb[k], isem[k])
        pltpu.async_copy(dst_hbm.at[wid, n], idb[k], isem[k])

    def wait_idx(k):
        pltpu.make_async_copy(src_hbm.at[wid, 0], isb[k], isem[k]).wait()
        pltpu.make_async_copy(dst_hbm.at[wid, 0], idb[k], isem[k]).wait()

    pltpu.sync_copy(src_hbm.at[wid, 0], is0)
    pltpu.sync_copy(dst_hbm.at[wid, 0], id0)
    issue_idx(1, 1)
    issue_idx(2, 2)
    plsc.subcore_barrier()
    pltpu.async_copy(x_hbm.at[is0], rows0, gsem0)

    def body(t, carry):
        for b in range(4):
            i = 4 * t + b
            rb, rbn = b % 2, (b + 1) % 2
            bn, bp = (b + 1) % 4, (b + 3) % 4
            pltpu.make_async_copy(x_hbm.at[isb[b]], rows[rb],
                                  gsem[rb]).wait()
            wait_idx(bn)
            pltpu.async_copy(x_hbm.at[isb[bn]], rows[rbn], gsem[rbn])
            pltpu.sync_copy(rows[rb], acc_sh.at[idb[b]], add=True)
            nx3 = jnp.minimum(i + 3, NCHUNKS - 1)
            issue_idx(nx3, bp)
        return carry

    lax.fori_loop(0, NCHUNKS // 4, body, 0)
    wait_gather(0)
    wait_idx(1)
    wait_idx(2)
    plsc.subcore_barrier()
    pltpu.sync_copy(acc_sh.at[pl.ds(r0, ROWS_PER_TILE)],
                    out_hbm.at[c, pl.ds(r0, ROWS_PER_TILE)])


_sc_agg = functools.partial(
    pl.kernel,
    mesh=plsc.VectorSubcoreMesh(core_axis_name="c", subcore_axis_name="s"),
    out_type=jax.ShapeDtypeStruct((NC, N_PAD, D), jnp.float32),
    scratch_types=[
        pltpu.VMEM_SHARED((N_PAD, D), jnp.float32),
        pltpu.VMEM((CHUNK, D), jnp.float32),
        pltpu.VMEM((CHUNK, D), jnp.float32),
        pltpu.VMEM((CHUNK,), jnp.int32),
        pltpu.VMEM((CHUNK,), jnp.int32),
        pltpu.VMEM((CHUNK,), jnp.int32),
        pltpu.VMEM((CHUNK,), jnp.int32),
        pltpu.VMEM((CHUNK,), jnp.int32),
        pltpu.VMEM((CHUNK,), jnp.int32),
        pltpu.VMEM((CHUNK,), jnp.int32),
        pltpu.VMEM((CHUNK,), jnp.int32),
        pltpu.SemaphoreType.DMA,
        pltpu.SemaphoreType.DMA,
        pltpu.SemaphoreType.DMA,
        pltpu.SemaphoreType.DMA,
        pltpu.SemaphoreType.DMA,
        pltpu.SemaphoreType.DMA,
    ],
)(_sc_agg_body)


BLK = 1024


def _tc_linear_body(p_ref, w_ref, b_ref, o_ref):
    agg = p_ref[0] + p_ref[1]
    o_ref[...] = lax.dot_general(
        agg, w_ref[...], (((1,), (1,)), ((), ())),
        preferred_element_type=jnp.float32) + b_ref[...]


def _tc_linear(partials, W, b):
    return pl.pallas_call(
        _tc_linear_body,
        grid=(N_PAD // BLK,),
        in_specs=[
            pl.BlockSpec((NC, BLK, D), lambda i: (0, i, 0)),
            pl.BlockSpec((D, D), lambda i: (0, 0)),
            pl.BlockSpec((1, D), lambda i: (0, 0)),
        ],
        out_specs=pl.BlockSpec((BLK, D), lambda i: (i, 0)),
        out_shape=jax.ShapeDtypeStruct((N_PAD, D), jnp.float32),
    )(partials, W, b.reshape(1, D))


def kernel(x, edge_index, W, b):
    src = edge_index[0].astype(jnp.int32)
    dst = edge_index[1].astype(jnp.int32)
    pad_src = jnp.zeros((NW, PAD_TILE), jnp.int32)
    pad_dst = jnp.broadcast_to(
        N_NODES + jnp.arange(PAD_TILE, dtype=jnp.int32), (NW, PAD_TILE))
    src3 = jnp.concatenate([src.reshape(NW, E_TILE), pad_src],
                           axis=1).reshape(NW, NCHUNKS, CHUNK)
    dst3 = jnp.concatenate([dst.reshape(NW, E_TILE), pad_dst],
                           axis=1).reshape(NW, NCHUNKS, CHUNK)
    zero = jnp.zeros((N_PAD, D), jnp.float32)
    partials = _sc_agg(x, src3, dst3, zero)
    return _tc_linear(partials, W, b)[:N_NODES]


# speedup vs baseline: 2.3927x; 1.7323x over previous
import functools

import jax
import jax.numpy as jnp
from jax import lax
from jax.experimental import pallas as pl
from jax.experimental.pallas import tpu as pltpu
from jax.experimental.pallas import tpu_sc as plsc

N_NODES = 10000
N_PAD = 10240
D = 128
N_EDGES = 320000
NC = 2
NS = 16
EDGES_PER_TILE = N_EDGES // (NC * NS)
CHUNK = 80
NCHUNKS = EDGES_PER_TILE // CHUNK
ROWS_PER_TILE = N_PAD // NS


def _sc_agg_body(x_hbm, src_hbm, dst_hbm, zero_hbm, out_hbm,
                 acc_sh, idx_s, idx_d, rows, sem):
    c = lax.axis_index("c")
    s = lax.axis_index("s")
    r0 = s * ROWS_PER_TILE
    pltpu.sync_copy(zero_hbm.at[pl.ds(r0, ROWS_PER_TILE)],
                    acc_sh.at[pl.ds(r0, ROWS_PER_TILE)])
    plsc.subcore_barrier()

    base = (c * NS + s) * EDGES_PER_TILE

    def body(i, carry):
        off = base + i * CHUNK
        pltpu.sync_copy(src_hbm.at[pl.ds(off, CHUNK)], idx_s)
        pltpu.sync_copy(dst_hbm.at[pl.ds(off, CHUNK)], idx_d)
        pltpu.async_copy(x_hbm.at[idx_s], rows, sem).wait()
        pltpu.sync_copy(rows, acc_sh.at[idx_d], add=True)
        return carry

    lax.fori_loop(0, NCHUNKS, body, 0)
    plsc.subcore_barrier()
    pltpu.sync_copy(acc_sh.at[pl.ds(r0, ROWS_PER_TILE)],
                    out_hbm.at[c, pl.ds(r0, ROWS_PER_TILE)])


_sc_agg = functools.partial(
    pl.kernel,
    mesh=plsc.VectorSubcoreMesh(core_axis_name="c", subcore_axis_name="s"),
    out_type=jax.ShapeDtypeStruct((NC, N_PAD, D), jnp.float32),
    scratch_types=[
        pltpu.VMEM_SHARED((N_PAD, D), jnp.float32),
        pltpu.VMEM((CHUNK,), jnp.int32),
        pltpu.VMEM((CHUNK,), jnp.int32),
        pltpu.VMEM((CHUNK, D), jnp.float32),
        pltpu.SemaphoreType.DMA,
    ],
)(_sc_agg_body)


BLK = 1024


def _tc_linear_body(p_ref, w_ref, b_ref, o_ref):
    agg = p_ref[0] + p_ref[1]
    o_ref[...] = lax.dot_general(
        agg, w_ref[...], (((1,), (1,)), ((), ())),
        preferred_element_type=jnp.float32) + b_ref[...]


def _tc_linear(partials, W, b):
    return pl.pallas_call(
        _tc_linear_body,
        grid=(N_PAD // BLK,),
        in_specs=[
            pl.BlockSpec((NC, BLK, D), lambda i: (0, i, 0)),
            pl.BlockSpec((D, D), lambda i: (0, 0)),
            pl.BlockSpec((1, D), lambda i: (0, 0)),
        ],
        out_specs=pl.BlockSpec((BLK, D), lambda i: (i, 0)),
        out_shape=jax.ShapeDtypeStruct((N_PAD, D), jnp.float32),
    )(partials, W, b.reshape(1, D))


def kernel(x, edge_index, W, b):
    src = edge_index[0].astype(jnp.int32)
    dst = edge_index[1].astype(jnp.int32)
    zero = jnp.zeros((N_PAD, D), jnp.float32)
    partials = _sc_agg(x, src, dst, zero)
    return _tc_linear(partials, W, b)[:N_NODES]


# speedup vs baseline: 3.7363x; 1.5616x over previous
import functools

import jax
import jax.numpy as jnp
from jax import lax
from jax.experimental import pallas as pl
from jax.experimental.pallas import tpu as pltpu
from jax.experimental.pallas import tpu_sc as plsc

N_NODES = 10000
N_PAD = 10240
D = 128
N_EDGES = 320000
NC = 2
NS = 16
EDGES_PER_TILE = N_EDGES // (NC * NS)
CHUNK = 80
NCHUNKS = EDGES_PER_TILE // CHUNK
ROWS_PER_TILE = N_PAD // NS


def _sc_agg_body(x_hbm, src_hbm, dst_hbm, zero_hbm, out_hbm,
                 acc_sh, is0, is1, id0, id1, rows0, rows1, gsem0, gsem1):
    c = lax.axis_index("c")
    s = lax.axis_index("s")
    r0 = s * ROWS_PER_TILE
    pltpu.sync_copy(zero_hbm.at[pl.ds(r0, ROWS_PER_TILE)],
                    acc_sh.at[pl.ds(r0, ROWS_PER_TILE)])
    plsc.subcore_barrier()

    base = (c * NS + s) * EDGES_PER_TILE
    isb = (is0, is1)
    idb = (id0, id1)
    rows = (rows0, rows1)
    gsem = (gsem0, gsem1)

    pltpu.sync_copy(src_hbm.at[pl.ds(base, CHUNK)], is0)
    pltpu.sync_copy(dst_hbm.at[pl.ds(base, CHUNK)], id0)
    pltpu.async_copy(x_hbm.at[is0], rows0, gsem0)

    def body(t, carry):
        for b in range(2):
            i = 2 * t + b
            bn = 1 - b
            off = base + (i + 1) * CHUNK
            pltpu.sync_copy(src_hbm.at[pl.ds(off, CHUNK)], isb[bn])
            pltpu.sync_copy(dst_hbm.at[pl.ds(off, CHUNK)], idb[bn])
            pltpu.async_copy(x_hbm.at[isb[bn]], rows[bn], gsem[bn])
            pltpu.make_async_copy(x_hbm.at[isb[b]], rows[b],
                                  gsem[b]).wait()
            pltpu.sync_copy(rows[b], acc_sh.at[idb[b]], add=True)
        return carry

    lax.fori_loop(0, (NCHUNKS - 1) // 2, body, 0)
    pltpu.make_async_copy(x_hbm.at[is0], rows0, gsem0).wait()
    pltpu.sync_copy(rows0, acc_sh.at[id0], add=True)
    plsc.subcore_barrier()
    pltpu.sync_copy(acc_sh.at[pl.ds(r0, ROWS_PER_TILE)],
                    out_hbm.at[c, pl.ds(r0, ROWS_PER_TILE)])


_sc_agg = functools.partial(
    pl.kernel,
    mesh=plsc.VectorSubcoreMesh(core_axis_name="c", subcore_axis_name="s"),
    out_type=jax.ShapeDtypeStruct((NC, N_PAD, D), jnp.float32),
    scratch_types=[
        pltpu.VMEM_SHARED((N_PAD, D), jnp.float32),
        pltpu.VMEM((CHUNK,), jnp.int32),
        pltpu.VMEM((CHUNK,), jnp.int32),
        pltpu.VMEM((CHUNK,), jnp.int32),
        pltpu.VMEM((CHUNK,), jnp.int32),
        pltpu.VMEM((CHUNK, D), jnp.float32),
        pltpu.VMEM((CHUNK, D), jnp.float32),
        pltpu.SemaphoreType.DMA,
        pltpu.SemaphoreType.DMA,
    ],
)(_sc_agg_body)


BLK = 1024


def _tc_linear_body(p_ref, w_ref, b_ref, o_ref):
    agg = p_ref[0] + p_ref[1]
    o_ref[...] = lax.dot_general(
        agg, w_ref[...], (((1,), (1,)), ((), ())),
        preferred_element_type=jnp.float32) + b_ref[...]


def _tc_linear(partials, W, b):
    return pl.pallas_call(
        _tc_linear_body,
        grid=(N_PAD // BLK,),
        in_specs=[
            pl.BlockSpec((NC, BLK, D), lambda i: (0, i, 0)),
            pl.BlockSpec((D, D), lambda i: (0, 0)),
            pl.BlockSpec((1, D), lambda i: (0, 0)),
        ],
        out_specs=pl.BlockSpec((BLK, D), lambda i: (i, 0)),
        out_shape=jax.ShapeDtypeStruct((N_PAD, D), jnp.float32),
    )(partials, W, b.reshape(1, D))


def kernel(x, edge_index, W, b):
    src = edge_index[0].astype(jnp.int32)
    dst = edge_index[1].astype(jnp.int32)
    zero = jnp.zeros((N_PAD, D), jnp.float32)
    partials = _sc_agg(x, src, dst, zero)
    return _tc_linear(partials, W, b)[:N_NODES]
